# baseline (speedup 1.0000x reference)
import functools

import jax
import jax.numpy as jnp
from jax import lax
from jax.experimental import pallas as pl
from jax.experimental.pallas import tpu as pltpu
from jax.experimental.pallas import tpu_sc as plsc

N_NODES = 10000
N_EDGES = 320000
D_IN = 128
D_OUT = 128

NC = 2
NS = 16
DH = D_IN // NC
DW = DH
CHUNK = 128
NPAD = 10112
ROWS_PER_TILE = NPAD // NS
NB = 5
NCHUNK = 160
NR = NCHUNK // NB
EPAD = NS * NCHUNK * CHUNK
HALF = NCHUNK // 2
DEGPAD = 640


def _sc_segment_sum(hext, src3, dst3):
  mesh = plsc.VectorSubcoreMesh(core_axis_name="c", subcore_axis_name="s")

  @functools.partial(
      pl.kernel,
      mesh=mesh,
      compiler_params=pltpu.CompilerParams(use_tc_tiling_on_sc=False),
      out_type=[
          jax.ShapeDtypeStruct((NC, NPAD, DW), jnp.float32),
          jax.ShapeDtypeStruct((NC, NS, ROWS_PER_TILE), jnp.float32),
      ],
      scratch_types=(
          [
              pltpu.VMEM((2, NCHUNK, CHUNK), jnp.int32),
              pltpu.VMEM((NB * CHUNK, DW), jnp.float32),
              pltpu.VMEM((DEGPAD + CHUNK,), jnp.float32),
              pltpu.VMEM_SHARED((NPAD, DW), jnp.float32),
              pltpu.VMEM_SHARED((NPAD,), jnp.float32),
          ]
          + [pltpu.SemaphoreType.DMA] * (2 * NB + 1)
      ),
  )
  def k(h_hbm, src_hbm, dst_hbm, sum_out, deg_out,
        idxbuf, rowsbuf, fbuf, acc_sh, degacc_sh, *sems):
    ones_v = fbuf.at[pl.ds(DEGPAD, CHUNK)]
    degbuf = fbuf.at[pl.ds(0, DEGPAD)]
    rows = [rowsbuf.at[pl.ds(b * CHUNK, CHUNK)] for b in range(NB)]
    gsem = sems[:NB]
    ssem = sems[NB:2 * NB]
    dsem = sems[2 * NB]
    c = lax.axis_index("c")
    s = lax.axis_index("s")
    base = s * ROWS_PER_TILE

    zeros16 = jnp.zeros((16,), jnp.float32)

    pltpu.sync_copy(src_hbm.at[s], idxbuf.at[0])
    pltpu.sync_copy(dst_hbm.at[s], idxbuf.at[1])

    cvec = jnp.broadcast_to(c, (16,))
    def xf(i, _):
      jj = i // (CHUNK // 16)
      kk = i % (CHUNK // 16)
      sl = pl.ds(kk * 16, 16)
      idxbuf[0, jj, sl] = idxbuf[0, jj, sl] * 2 + cvec
      return 0
    lax.fori_loop(0, NCHUNK * (CHUNK // 16), xf, 0)

    def zrow(i, _):
      for kk in range(DW // 16):
        rowsbuf[i, pl.ds(kk * 16, 16)] = zeros16
      return 0
    lax.fori_loop(0, CHUNK, zrow, 0)

    def zfb(i, _):
      fbuf[pl.ds(i * 16, 16)] = zeros16
      return 0
    lax.fori_loop(0, DEGPAD // 16, zfb, 0)

    def zone(i, _):
      fbuf[pl.ds(DEGPAD + i * 16, 16)] = jnp.ones((16,), jnp.float32)
      return 0
    lax.fori_loop(0, CHUNK // 16, zone, 0)

    nfull = ROWS_PER_TILE // CHUNK
    rem = ROWS_PER_TILE - nfull * CHUNK
    for kk in range(nfull):
      pltpu.sync_copy(rowsbuf.at[pl.ds(0, CHUNK)],
                      acc_sh.at[pl.ds(base + kk * CHUNK, CHUNK)])
    pltpu.sync_copy(rowsbuf.at[pl.ds(0, rem)],
                    acc_sh.at[pl.ds(base + nfull * CHUNK, rem)])
    pltpu.sync_copy(degbuf.at[pl.ds(0, ROWS_PER_TILE)],
                    degacc_sh.at[pl.ds(base, ROWS_PER_TILE)])

    plsc.subcore_barrier()

    for b in range(NB):
      pltpu.async_copy(h_hbm.at[idxbuf.at[0, b]], rows[b], gsem[b])

    def rnd(g, _):
      j0 = g * NB
      for b in range(NB):
        pltpu.make_async_copy(h_hbm.at[pl.ds(0, CHUNK)], rows[b],
                              gsem[b]).wait()
        pltpu.async_copy(rows[b], acc_sh.at[idxbuf.at[1, j0 + b]], ssem[b],
                         add=True)
        @pl.when(jnp.where(c == 0, j0 + b < HALF, j0 + b >= HALF))
        def _():
          pltpu.async_copy(ones_v, degacc_sh.at[idxbuf.at[1, j0 + b]], dsem,
                           add=True)
      for b in range(NB):
        pltpu.make_async_copy(rows[b], acc_sh.at[pl.ds(0, CHUNK)],
                              ssem[b]).wait()
        @pl.when(g < NR - 1)
        def _():
          pltpu.async_copy(h_hbm.at[idxbuf.at[0, j0 + NB + b]], rows[b],
                           gsem[b])
      for b in range(NB):
        @pl.when(jnp.where(c == 0, j0 + b < HALF, j0 + b >= HALF))
        def _():
          pltpu.make_async_copy(ones_v, degacc_sh.at[pl.ds(0, CHUNK)],
                                dsem).wait()
      return 0
    lax.fori_loop(0, NR, rnd, 0)

    plsc.subcore_barrier()

    for kk in range(nfull):
      pltpu.sync_copy(acc_sh.at[pl.ds(base + kk * CHUNK, CHUNK)],
                      rowsbuf.at[pl.ds(0, CHUNK)])
      pltpu.sync_copy(rowsbuf.at[pl.ds(0, CHUNK)],
                      sum_out.at[c, pl.ds(base + kk * CHUNK, CHUNK)])
    pltpu.sync_copy(acc_sh.at[pl.ds(base + nfull * CHUNK, rem)],
                    rowsbuf.at[pl.ds(0, rem)])
    pltpu.sync_copy(rowsbuf.at[pl.ds(0, rem)],
                    sum_out.at[c, pl.ds(base + nfull * CHUNK, rem)])
    pltpu.sync_copy(degacc_sh.at[pl.ds(base, ROWS_PER_TILE)],
                    degbuf.at[pl.ds(0, ROWS_PER_TILE)])
    pltpu.sync_copy(degbuf.at[pl.ds(0, ROWS_PER_TILE)], deg_out.at[c, s])

  return k(hext, src3, dst3)


def _tc_combine(h, sums, degs, wt, b2):
  R = 2000
  grid = (N_NODES // R,)

  def body(h_ref, p_ref, d_ref, wt_ref, b_ref, o_ref):
    deg = d_ref[0] + d_ref[1]
    inv = 1.0 / jnp.maximum(deg, 1.0)
    hn = jnp.concatenate([p_ref[0, :, :DH], p_ref[1, :, :DH]], axis=1) * inv
    acc = jnp.dot(h_ref[...], wt_ref[0:D_IN, :],
                  preferred_element_type=jnp.float32)
    acc += jnp.dot(hn, wt_ref[D_IN:2 * D_IN, :],
                   preferred_element_type=jnp.float32)
    o_ref[...] = acc + b_ref[...]

  return pl.pallas_call(
      body,
      grid=grid,
      in_specs=[
          pl.BlockSpec((R, D_IN), lambda i: (i, 0)),
          pl.BlockSpec((NC, R, DW), lambda i: (0, i, 0)),
          pl.BlockSpec((NC, R, 1), lambda i: (0, i, 0)),
          pl.BlockSpec((2 * D_IN, D_OUT), lambda i: (0, 0)),
          pl.BlockSpec((1, D_OUT), lambda i: (0, 0)),
      ],
      out_specs=pl.BlockSpec((R, D_OUT), lambda i: (i, 0)),
      out_shape=jax.ShapeDtypeStruct((N_NODES, D_OUT), jnp.float32),
  )(h, sums, degs, wt, b2)


def kernel(h, edge_index, W, b):
  src = edge_index[0].astype(jnp.int32)
  dst = edge_index[1].astype(jnp.int32)
  npad_e = EPAD - N_EDGES
  src = jnp.concatenate([src, jnp.zeros((npad_e,), jnp.int32)])
  dst = jnp.concatenate([dst, jnp.full((npad_e,), N_NODES, jnp.int32)])
  src3 = src.reshape(NS, NCHUNK, CHUNK)
  dst3 = dst.reshape(NS, NCHUNK, CHUNK)
  hext = h.reshape(2 * N_NODES, DH)

  sums, degs = _sc_segment_sum(hext, src3, dst3)
  degs = degs.reshape(NC, NPAD, 1)

  wt = W.T
  b2 = b.reshape(1, D_OUT)
  return _tc_combine(h, sums, degs, wt, b2)


# speedup vs baseline: 1.0098x; 1.0098x over previous
import functools

import jax
import jax.numpy as jnp
from jax import lax
from jax.experimental import pallas as pl
from jax.experimental.pallas import tpu as pltpu
from jax.experimental.pallas import tpu_sc as plsc

N_NODES = 10000
N_EDGES = 320000
D_IN = 128
D_OUT = 128

NC = 2
NS = 16
DH = D_IN // NC
DW = DH
CHUNK = 128
NPAD = 10112
ROWS_PER_TILE = NPAD // NS
NB = 5
NCHUNK = 160
NR = NCHUNK // NB
EPAD = NS * NCHUNK * CHUNK
HALF = NCHUNK // 2
DEGPAD = 640


def _sc_segment_sum(hext, src3, dst3):
  mesh = plsc.VectorSubcoreMesh(core_axis_name="c", subcore_axis_name="s")

  @functools.partial(
      pl.kernel,
      mesh=mesh,
      compiler_params=pltpu.CompilerParams(use_tc_tiling_on_sc=False),
      out_type=[
          jax.ShapeDtypeStruct((NC, NPAD, DW), jnp.float32),
          jax.ShapeDtypeStruct((NC, NS, ROWS_PER_TILE), jnp.float32),
      ],
      scratch_types=(
          [
              pltpu.VMEM((2, NCHUNK, CHUNK), jnp.int32),
              pltpu.VMEM((NB * CHUNK, DW), jnp.float32),
              pltpu.VMEM((DEGPAD + CHUNK,), jnp.float32),
              pltpu.VMEM_SHARED((NPAD, DW), jnp.float32),
              pltpu.VMEM_SHARED((NPAD,), jnp.float32),
          ]
          + [pltpu.SemaphoreType.DMA] * (2 * NB + 1)
      ),
  )
  def k(h_hbm, src_hbm, dst_hbm, sum_out, deg_out,
        idxbuf, rowsbuf, fbuf, acc_sh, degacc_sh, *sems):
    ones_v = fbuf.at[pl.ds(DEGPAD, CHUNK)]
    degbuf = fbuf.at[pl.ds(0, DEGPAD)]
    rows = [rowsbuf.at[pl.ds(b * CHUNK, CHUNK)] for b in range(NB)]
    gsem = sems[:NB]
    ssem = sems[NB:2 * NB]
    dsem = sems[2 * NB]
    c = lax.axis_index("c")
    s = lax.axis_index("s")
    base = s * ROWS_PER_TILE

    zeros16 = jnp.zeros((16,), jnp.float32)

    pltpu.sync_copy(src_hbm.at[c, s], idxbuf.at[0])
    pltpu.sync_copy(dst_hbm.at[s], idxbuf.at[1])

    def zrow(i, _):
      for kk in range(DW // 16):
        rowsbuf[i, pl.ds(kk * 16, 16)] = zeros16
      return 0
    lax.fori_loop(0, CHUNK, zrow, 0)

    def zfb(i, _):
      fbuf[pl.ds(i * 16, 16)] = zeros16
      return 0
    lax.fori_loop(0, DEGPAD // 16, zfb, 0)

    def zone(i, _):
      fbuf[pl.ds(DEGPAD + i * 16, 16)] = jnp.ones((16,), jnp.float32)
      return 0
    lax.fori_loop(0, CHUNK // 16, zone, 0)

    nfull = ROWS_PER_TILE // CHUNK
    rem = ROWS_PER_TILE - nfull * CHUNK
    for kk in range(nfull):
      pltpu.sync_copy(rowsbuf.at[pl.ds(0, CHUNK)],
                      acc_sh.at[pl.ds(base + kk * CHUNK, CHUNK)])
    pltpu.sync_copy(rowsbuf.at[pl.ds(0, rem)],
                    acc_sh.at[pl.ds(base + nfull * CHUNK, rem)])
    pltpu.sync_copy(degbuf.at[pl.ds(0, ROWS_PER_TILE)],
                    degacc_sh.at[pl.ds(base, ROWS_PER_TILE)])

    plsc.subcore_barrier()

    for b in range(NB):
      pltpu.async_copy(h_hbm.at[idxbuf.at[0, b]], rows[b], gsem[b])

    def rnd(g, _):
      j0 = g * NB
      for b in range(NB):
        pltpu.make_async_copy(h_hbm.at[pl.ds(0, CHUNK)], rows[b],
                              gsem[b]).wait()
        pltpu.async_copy(rows[b], acc_sh.at[idxbuf.at[1, j0 + b]], ssem[b],
                         add=True)
        pltpu.async_copy(ones_v, degacc_sh.at[idxbuf.at[1, j0 + b]], dsem,
                         add=True)
      for b in range(NB):
        pltpu.make_async_copy(rows[b], acc_sh.at[pl.ds(0, CHUNK)],
                              ssem[b]).wait()
        @pl.when(g < NR - 1)
        def _():
          pltpu.async_copy(h_hbm.at[idxbuf.at[0, j0 + NB + b]], rows[b],
                           gsem[b])
      for b in range(NB):
        pltpu.make_async_copy(ones_v, degacc_sh.at[pl.ds(0, CHUNK)],
                              dsem).wait()
      return 0
    lax.fori_loop(0, NR, rnd, 0)

    plsc.subcore_barrier()

    for kk in range(nfull):
      pltpu.sync_copy(acc_sh.at[pl.ds(base + kk * CHUNK, CHUNK)],
                      rowsbuf.at[pl.ds(0, CHUNK)])
      pltpu.sync_copy(rowsbuf.at[pl.ds(0, CHUNK)],
                      sum_out.at[c, pl.ds(base + kk * CHUNK, CHUNK)])
    pltpu.sync_copy(acc_sh.at[pl.ds(base + nfull * CHUNK, rem)],
                    rowsbuf.at[pl.ds(0, rem)])
    pltpu.sync_copy(rowsbuf.at[pl.ds(0, rem)],
                    sum_out.at[c, pl.ds(base + nfull * CHUNK, rem)])
    pltpu.sync_copy(degacc_sh.at[pl.ds(base, ROWS_PER_TILE)],
                    degbuf.at[pl.ds(0, ROWS_PER_TILE)])
    pltpu.sync_copy(degbuf.at[pl.ds(0, ROWS_PER_TILE)], deg_out.at[c, s])

  return k(hext, src3, dst3)


def _tc_combine(h, sums, degs, wt, b2):
  R = 2000
  grid = (N_NODES // R,)

  def body(h_ref, p_ref, d_ref, wt_ref, b_ref, o_ref):
    deg = (d_ref[0] + d_ref[1]) * 0.5
    inv = 1.0 / jnp.maximum(deg, 1.0)
    hn = jnp.concatenate([p_ref[0, :, :DH], p_ref[1, :, :DH]], axis=1) * inv
    acc = jnp.dot(h_ref[...], wt_ref[0:D_IN, :],
                  preferred_element_type=jnp.float32)
    acc += jnp.dot(hn, wt_ref[D_IN:2 * D_IN, :],
                   preferred_element_type=jnp.float32)
    o_ref[...] = acc + b_ref[...]

  return pl.pallas_call(
      body,
      grid=grid,
      in_specs=[
          pl.BlockSpec((R, D_IN), lambda i: (i, 0)),
          pl.BlockSpec((NC, R, DW), lambda i: (0, i, 0)),
          pl.BlockSpec((NC, R, 1), lambda i: (0, i, 0)),
          pl.BlockSpec((2 * D_IN, D_OUT), lambda i: (0, 0)),
          pl.BlockSpec((1, D_OUT), lambda i: (0, 0)),
      ],
      out_specs=pl.BlockSpec((R, D_OUT), lambda i: (i, 0)),
      out_shape=jax.ShapeDtypeStruct((N_NODES, D_OUT), jnp.float32),
  )(h, sums, degs, wt, b2)


def kernel(h, edge_index, W, b):
  src = edge_index[0].astype(jnp.int32)
  dst = edge_index[1].astype(jnp.int32)
  npad_e = EPAD - N_EDGES
  src = jnp.concatenate([src, jnp.zeros((npad_e,), jnp.int32)])
  dst = jnp.concatenate([dst, jnp.full((npad_e,), N_NODES, jnp.int32)])
  src3 = jnp.stack([src * 2, src * 2 + 1]).reshape(NC, NS, NCHUNK, CHUNK)
  dst3 = dst.reshape(NS, NCHUNK, CHUNK)
  hext = h.reshape(2 * N_NODES, DH)

  sums, degs = _sc_segment_sum(hext, src3, dst3)
  degs = degs.reshape(NC, NPAD, 1)

  wt = W.T
  b2 = b.reshape(1, D_OUT)
  return _tc_combine(h, sums, degs, wt, b2)


# speedup vs baseline: 1.2213x; 1.2095x over previous
import functools

import jax
import jax.numpy as jnp
from jax import lax
from jax.experimental import pallas as pl
from jax.experimental.pallas import tpu as pltpu
from jax.experimental.pallas import tpu_sc as plsc

N_NODES = 10000
N_EDGES = 320000
D_IN = 128
D_OUT = 128

NC = 2
NS = 16
DH = D_IN // NC
DW = DH
CHUNK = 128
NPAD = 10112
ROWS_PER_TILE = NPAD // NS
NB = 5
NCHUNK = 160
NR = NCHUNK // NB
EPAD = NS * NCHUNK * CHUNK
HALF = NCHUNK // 2
DEGPAD = 640


def _sc_segment_sum(hext, src3, dst3):
  mesh = plsc.VectorSubcoreMesh(core_axis_name="c", subcore_axis_name="s")

  @functools.partial(
      pl.kernel,
      mesh=mesh,
      compiler_params=pltpu.CompilerParams(use_tc_tiling_on_sc=False),
      out_type=[
          jax.ShapeDtypeStruct((NC, NPAD, DW), jnp.float32),
          jax.ShapeDtypeStruct((NC, NS, ROWS_PER_TILE), jnp.float32),
      ],
      scratch_types=(
          [
              pltpu.VMEM((2, NCHUNK, CHUNK), jnp.int32),
              pltpu.VMEM((NB * CHUNK, DW), jnp.float32),
              pltpu.VMEM((DEGPAD + CHUNK,), jnp.float32),
              pltpu.VMEM_SHARED((NPAD, DW), jnp.float32),
              pltpu.VMEM_SHARED((NPAD,), jnp.float32),
          ]
          + [pltpu.SemaphoreType.DMA] * (2 * NB + 1)
      ),
  )
  def k(h_hbm, src_hbm, dst_hbm, sum_out, deg_out,
        idxbuf, rowsbuf, fbuf, acc_sh, degacc_sh, *sems):
    ones_v = fbuf.at[pl.ds(DEGPAD, CHUNK)]
    degbuf = fbuf.at[pl.ds(0, DEGPAD)]
    rows = [rowsbuf.at[pl.ds(b * CHUNK, CHUNK)] for b in range(NB)]
    gsem = sems[:NB]
    ssem = sems[NB:2 * NB]
    dsem = sems[2 * NB]
    c = lax.axis_index("c")
    s = lax.axis_index("s")
    base = s * ROWS_PER_TILE

    zeros16 = jnp.zeros((16,), jnp.float32)

    pltpu.sync_copy(src_hbm.at[c, s], idxbuf.at[0])
    pltpu.sync_copy(dst_hbm.at[s], idxbuf.at[1])

    def zrow(i, _):
      for kk in range(DW // 16):
        rowsbuf[i, pl.ds(kk * 16, 16)] = zeros16
      return 0
    lax.fori_loop(0, CHUNK, zrow, 0)

    def zfb(i, _):
      fbuf[pl.ds(i * 16, 16)] = zeros16
      return 0
    lax.fori_loop(0, DEGPAD // 16, zfb, 0)

    def zone(i, _):
      fbuf[pl.ds(DEGPAD + i * 16, 16)] = jnp.ones((16,), jnp.float32)
      return 0
    lax.fori_loop(0, CHUNK // 16, zone, 0)

    nfull = ROWS_PER_TILE // CHUNK
    rem = ROWS_PER_TILE - nfull * CHUNK
    for kk in range(nfull):
      pltpu.sync_copy(rowsbuf.at[pl.ds(0, CHUNK)],
                      acc_sh.at[pl.ds(base + kk * CHUNK, CHUNK)])
    pltpu.sync_copy(rowsbuf.at[pl.ds(0, rem)],
                    acc_sh.at[pl.ds(base + nfull * CHUNK, rem)])
    pltpu.sync_copy(degbuf.at[pl.ds(0, ROWS_PER_TILE)],
                    degacc_sh.at[pl.ds(base, ROWS_PER_TILE)])

    plsc.subcore_barrier()

    for b in range(NB):
      pltpu.async_copy(h_hbm.at[idxbuf.at[0, b]], rows[b], gsem[b])

    def rnd(g, _):
      j0 = g * NB
      for b in range(NB):
        pltpu.make_async_copy(h_hbm.at[pl.ds(0, CHUNK)], rows[b],
                              gsem[b]).wait()
        pltpu.async_copy(rows[b], acc_sh.at[idxbuf.at[1, j0 + b]], ssem[b],
                         add=True)
        pltpu.async_copy(ones_v, degacc_sh.at[idxbuf.at[1, j0 + b]], dsem,
                         add=True)
      for b in range(NB):
        pltpu.make_async_copy(rows[b], acc_sh.at[pl.ds(0, CHUNK)],
                              ssem[b]).wait()
        @pl.when(g < NR - 1)
        def _():
          pltpu.async_copy(h_hbm.at[idxbuf.at[0, j0 + NB + b]], rows[b],
                           gsem[b])
      for b in range(NB):
        pltpu.make_async_copy(ones_v, degacc_sh.at[pl.ds(0, CHUNK)],
                              dsem).wait()
      return 0
    lax.fori_loop(0, NR, rnd, 0)

    plsc.subcore_barrier()

    for kk in range(nfull):
      pltpu.sync_copy(acc_sh.at[pl.ds(base + kk * CHUNK, CHUNK)],
                      rowsbuf.at[pl.ds(0, CHUNK)])
      pltpu.sync_copy(rowsbuf.at[pl.ds(0, CHUNK)],
                      sum_out.at[c, pl.ds(base + kk * CHUNK, CHUNK)])
    pltpu.sync_copy(acc_sh.at[pl.ds(base + nfull * CHUNK, rem)],
                    rowsbuf.at[pl.ds(0, rem)])
    pltpu.sync_copy(rowsbuf.at[pl.ds(0, rem)],
                    sum_out.at[c, pl.ds(base + nfull * CHUNK, rem)])
    pltpu.sync_copy(degacc_sh.at[pl.ds(base, ROWS_PER_TILE)],
                    degbuf.at[pl.ds(0, ROWS_PER_TILE)])
    pltpu.sync_copy(degbuf.at[pl.ds(0, ROWS_PER_TILE)], deg_out.at[c, s])

  return k(hext, src3, dst3)


def _tc_combine(h, sums, degs, wt, b2):
  R = 2000
  grid = (N_NODES // R,)

  def body(h_ref, p_ref, d_ref, wt_ref, b_ref, o_ref):
    deg = (d_ref[0] + d_ref[1]) * 0.5
    inv = 1.0 / jnp.maximum(deg, 1.0)
    hn = jnp.concatenate([p_ref[0, :, :DH], p_ref[1, :, :DH]], axis=1) * inv
    acc = jnp.dot(h_ref[...], wt_ref[0:D_IN, :],
                  preferred_element_type=jnp.float32)
    acc += jnp.dot(hn, wt_ref[D_IN:2 * D_IN, :],
                   preferred_element_type=jnp.float32)
    o_ref[...] = acc + b_ref[...]

  return pl.pallas_call(
      body,
      grid=grid,
      in_specs=[
          pl.BlockSpec((R, D_IN), lambda i: (i, 0)),
          pl.BlockSpec((NC, R, DW), lambda i: (0, i, 0)),
          pl.BlockSpec((NC, R, 1), lambda i: (0, i, 0)),
          pl.BlockSpec((2 * D_IN, D_OUT), lambda i: (0, 0)),
          pl.BlockSpec((1, D_OUT), lambda i: (0, 0)),
      ],
      out_specs=pl.BlockSpec((R, D_OUT), lambda i: (i, 0)),
      out_shape=jax.ShapeDtypeStruct((N_NODES, D_OUT), jnp.float32),
  )(h, sums, degs, wt, b2)


def kernel(h, edge_index, W, b):
  src = edge_index[0].astype(jnp.int32)
  dst = edge_index[1].astype(jnp.int32)
  npad_e = EPAD - N_EDGES
  src = jnp.concatenate([src, jnp.zeros((npad_e,), jnp.int32)])
  dst = jnp.concatenate([dst, jnp.full((npad_e,), N_NODES, jnp.int32)])
  src3 = jnp.stack([src, src + N_NODES]).reshape(NC, NS, NCHUNK, CHUNK)
  dst3 = dst.reshape(NS, NCHUNK, CHUNK)
  hext = jnp.concatenate([h[:, :DH], h[:, DH:]], axis=0)

  sums, degs = _sc_segment_sum(hext, src3, dst3)
  degs = degs.reshape(NC, NPAD, 1)

  wt = W.T
  b2 = b.reshape(1, D_OUT)
  return _tc_combine(h, sums, degs, wt, b2)
